# Initial kernel scaffold; baseline (speedup 1.0000x reference)
import jax
import jax.numpy as jnp
from jax.experimental import pallas as pl


def kernel(x, adj_indices, adj_values, W, b):
    raise NotImplementedError("write your pallas kernel here")


# speedup vs baseline: 6.5163x; 6.5163x over previous
import functools

import jax
import jax.numpy as jnp
from jax import lax
from jax.experimental import pallas as pl
from jax.experimental.pallas import tpu as pltpu
from jax.experimental.pallas import tpu_sc as plsc

NC = 2
NS = 16
NW = NC * NS
LANES = 16


def _sc_aggregate(x, rows2d, cols2d, vals2d, n_nodes, d,
                  n_chunks, n_chunks_pad, c_edges):
    rps = (n_nodes // (8 * NS)) * 8
    rem_groups = (n_nodes - rps * NS) // 8
    assert n_nodes == rps * NS + rem_groups * 8
    zr = 48
    assert rps % zr == 0
    n_copies = rps // zr
    slab = 32
    assert n_chunks_pad % slab == 0

    mesh = plsc.VectorSubcoreMesh(core_axis_name="c", subcore_axis_name="s")

    @functools.partial(
        pl.kernel,
        mesh=mesh,
        out_type=jax.ShapeDtypeStruct((NC * n_nodes, d), jnp.float32),
        scratch_types=[
            pltpu.VMEM_SHARED((n_nodes, d), jnp.float32),
            pltpu.VMEM((slab, c_edges), jnp.int32),
            pltpu.VMEM((slab, c_edges), jnp.int32),
            pltpu.VMEM((slab, c_edges), jnp.float32),
            pltpu.VMEM((c_edges, d), jnp.float32),
            pltpu.VMEM((zr, d), jnp.float32),
        ],
    )
    def sc_agg(x_hbm, rows_hbm, cols_hbm, vals_hbm, out_hbm,
               acc, rbuf, cbuf, vbuf, gbuf, zbuf):
        c = lax.axis_index("c")
        s = lax.axis_index("s")
        wid = s * NC + c

        def zrow(i, carry):
            for k in range(d // LANES):
                zbuf[i, pl.ds(k * LANES, LANES)] = jnp.zeros(
                    (LANES,), jnp.float32)
            return carry
        lax.fori_loop(0, zr, zrow, 0)
        for t in range(n_copies):
            pltpu.sync_copy(zbuf, acc.at[pl.ds(s * rps + t * zr, zr)])
        for g in range(rem_groups):
            @pl.when(s == g)
            def _():
                pltpu.sync_copy(
                    zbuf.at[pl.ds(0, 8)],
                    acc.at[pl.ds(NS * rps + g * 8, 8)])

        plsc.subcore_barrier()

        for t in range(n_chunks_pad // slab):
            n_real = min(slab, n_chunks - t * slab)
            if n_real <= 0:
                break
            base = wid * n_chunks_pad + t * slab
            pltpu.sync_copy(rows_hbm.at[pl.ds(base, slab)], rbuf)
            pltpu.sync_copy(cols_hbm.at[pl.ds(base, slab)], cbuf)
            pltpu.sync_copy(vals_hbm.at[pl.ds(base, slab)], vbuf)

            def chunk_body(j, carry):
                pltpu.sync_copy(x_hbm.at[cbuf.at[j]], gbuf)

                def grp_body(g, carry2):
                    vv = vbuf[j, pl.ds(g * LANES, LANES)]
                    for e in range(LANES):
                        v = vv[e]
                        row = g * LANES + e
                        for k in range(d // LANES):
                            sl = pl.ds(k * LANES, LANES)
                            gbuf[row, sl] = gbuf[row, sl] * v
                    return carry2
                lax.fori_loop(0, c_edges // LANES, grp_body, 0)

                pltpu.sync_copy(gbuf, acc.at[rbuf.at[j]], add=True)
                return carry
            lax.fori_loop(0, n_real, chunk_body, 0)

        plsc.subcore_barrier()

        base = c * n_nodes
        for t in range(n_copies):
            pltpu.sync_copy(acc.at[pl.ds(s * rps + t * zr, zr)],
                            out_hbm.at[pl.ds(base + s * rps + t * zr, zr)])
        for g in range(rem_groups):
            @pl.when(s == g)
            def _():
                pltpu.sync_copy(
                    acc.at[pl.ds(NS * rps + g * 8, 8)],
                    out_hbm.at[pl.ds(base + NS * rps + g * 8, 8)])

    return sc_agg(x, rows2d, cols2d, vals2d)


def _tc_transform(acc2, W, b2, n_nodes, d):
    blk = 1000
    assert n_nodes % blk == 0

    def tc_body(acc_ref, w_ref, b_ref, o_ref):
        agg = acc_ref[0] + acc_ref[1]
        o_ref[...] = lax.dot_general(
            agg, w_ref[...], (((1,), (1,)), ((), ())),
            preferred_element_type=jnp.float32) + b_ref[...]

    return pl.pallas_call(
        tc_body,
        grid=(n_nodes // blk,),
        in_specs=[
            pl.BlockSpec((2, blk, d), lambda i: (0, i, 0)),
            pl.BlockSpec((d, d), lambda i: (0, 0)),
            pl.BlockSpec((1, d), lambda i: (0, 0)),
        ],
        out_specs=pl.BlockSpec((blk, d), lambda i: (i, 0)),
        out_shape=jax.ShapeDtypeStruct((n_nodes, d), jnp.float32),
    )(acc2, W, b2)


def kernel(x, adj_indices, adj_values, W, b):
    n_nodes, d = x.shape
    n_edges = adj_values.shape[0]

    c_edges = 80
    assert n_edges % (NW * c_edges) == 0
    n_chunks = n_edges // (NW * c_edges)
    n_chunks_pad = -(-n_chunks // 8) * 8

    def slab(a):
        a = a.reshape(NW, n_chunks * c_edges)
        pad = n_chunks_pad * c_edges - n_chunks * c_edges
        a = jnp.pad(a, ((0, 0), (0, pad)))
        return a.reshape(NW * n_chunks_pad, c_edges)

    rows2d = slab(adj_indices[0])
    cols2d = slab(adj_indices[1])
    vals2d = slab(adj_values)

    partials = _sc_aggregate(x, rows2d, cols2d, vals2d,
                             n_nodes, d, n_chunks, n_chunks_pad, c_edges)
    acc2 = partials.reshape(2, n_nodes, d)
    return _tc_transform(acc2, W, b.reshape(1, d), n_nodes, d)


# speedup vs baseline: 7.5342x; 1.1562x over previous
import functools

import jax
import jax.numpy as jnp
from jax import lax
from jax.experimental import pallas as pl
from jax.experimental.pallas import tpu as pltpu
from jax.experimental.pallas import tpu_sc as plsc

NC = 2
NS = 16
NW = NC * NS
LANES = 16


def _sc_aggregate(x, rows2d, cols2d, vals2d, n_nodes, d,
                  n_chunks, n_chunks_pad, c_edges):
    rps = (n_nodes // (8 * NS)) * 8
    rem_groups = (n_nodes - rps * NS) // 8
    assert n_nodes == rps * NS + rem_groups * 8
    zr = 48
    assert rps % zr == 0
    n_copies = rps // zr
    slab = 32
    assert n_chunks_pad % slab == 0

    mesh = plsc.VectorSubcoreMesh(core_axis_name="c", subcore_axis_name="s")

    @functools.partial(
        pl.kernel,
        mesh=mesh,
        out_type=jax.ShapeDtypeStruct((NC * n_nodes, d), jnp.float32),
        scratch_types=[
            pltpu.VMEM_SHARED((n_nodes, d), jnp.float32),
            pltpu.VMEM((slab, c_edges), jnp.int32),
            pltpu.VMEM((slab, c_edges), jnp.int32),
            pltpu.VMEM((slab, c_edges), jnp.float32),
            pltpu.VMEM((c_edges, d), jnp.float32),
            pltpu.VMEM((c_edges, d), jnp.float32),
            pltpu.VMEM((zr, d), jnp.float32),
            pltpu.SemaphoreType.DMA,
            pltpu.SemaphoreType.DMA,
            pltpu.SemaphoreType.DMA,
            pltpu.SemaphoreType.DMA,
        ],
    )
    def sc_agg(x_hbm, rows_hbm, cols_hbm, vals_hbm, out_hbm,
               acc, rbuf, cbuf, vbuf, gbuf0, gbuf1, zbuf,
               gsem0, gsem1, ssem0, ssem1):
        c = lax.axis_index("c")
        s = lax.axis_index("s")
        wid = s * NC + c

        def zrow(i, carry):
            for k in range(d // LANES):
                zbuf[i, pl.ds(k * LANES, LANES)] = jnp.zeros(
                    (LANES,), jnp.float32)
            return carry
        lax.fori_loop(0, zr, zrow, 0)
        for t in range(n_copies):
            pltpu.sync_copy(zbuf, acc.at[pl.ds(s * rps + t * zr, zr)])
        for g in range(rem_groups):
            @pl.when(s == g)
            def _():
                pltpu.sync_copy(
                    zbuf.at[pl.ds(0, 8)],
                    acc.at[pl.ds(NS * rps + g * 8, 8)])

        plsc.subcore_barrier()

        def start_gather(j, buf, sem):
            pltpu.async_copy(x_hbm.at[cbuf.at[j]], buf, sem)

        def wait_gather(buf, sem):
            pltpu.make_async_copy(x_hbm.at[pl.ds(0, c_edges)], buf, sem).wait()

        def start_scatter(j, buf, sem):
            pltpu.async_copy(buf, acc.at[rbuf.at[j]], sem, add=True)

        def wait_scatter(buf, sem):
            pltpu.make_async_copy(buf, acc.at[pl.ds(0, c_edges)], sem).wait()

        def scale(j, buf):
            def grp_body(g, carry2):
                vv = vbuf[j, pl.ds(g * LANES, LANES)]
                for e in range(LANES):
                    v = vv[e]
                    row = g * LANES + e
                    for k in range(d // LANES):
                        sl = pl.ds(k * LANES, LANES)
                        buf[row, sl] = buf[row, sl] * v
                return carry2
            lax.fori_loop(0, c_edges // LANES, grp_body, 0)

        for t in range(n_chunks_pad // slab):
            cnt = min(slab, n_chunks - t * slab)
            if cnt <= 0:
                break
            base = wid * n_chunks_pad + t * slab
            pltpu.sync_copy(rows_hbm.at[pl.ds(base, slab)], rbuf)
            pltpu.sync_copy(cols_hbm.at[pl.ds(base, slab)], cbuf)
            pltpu.sync_copy(vals_hbm.at[pl.ds(base, slab)], vbuf)

            n_pairs = cnt // 2
            assert n_pairs >= 2

            start_gather(0, gbuf0, gsem0)
            wait_gather(gbuf0, gsem0)
            scale(0, gbuf0)
            start_gather(1, gbuf1, gsem1)
            start_scatter(0, gbuf0, ssem0)
            wait_gather(gbuf1, gsem1)
            scale(1, gbuf1)
            wait_scatter(gbuf0, ssem0)
            start_gather(2, gbuf0, gsem0)
            start_scatter(1, gbuf1, ssem1)

            def pair_body(i, carry):
                wait_gather(gbuf0, gsem0)
                scale(2 * i, gbuf0)
                wait_scatter(gbuf1, ssem1)
                start_gather(2 * i + 1, gbuf1, gsem1)
                start_scatter(2 * i, gbuf0, ssem0)
                wait_gather(gbuf1, gsem1)
                scale(2 * i + 1, gbuf1)
                wait_scatter(gbuf0, ssem0)
                start_gather(2 * i + 2, gbuf0, gsem0)
                start_scatter(2 * i + 1, gbuf1, ssem1)
                return carry
            lax.fori_loop(1, n_pairs - 1, pair_body, 0)

            last = 2 * (n_pairs - 1)
            wait_gather(gbuf0, gsem0)
            scale(last, gbuf0)
            wait_scatter(gbuf1, ssem1)
            start_gather(last + 1, gbuf1, gsem1)
            start_scatter(last, gbuf0, ssem0)
            wait_gather(gbuf1, gsem1)
            scale(last + 1, gbuf1)
            start_scatter(last + 1, gbuf1, ssem1)
            wait_scatter(gbuf0, ssem0)
            wait_scatter(gbuf1, ssem1)

            if cnt % 2 == 1:
                j = cnt - 1
                pltpu.sync_copy(x_hbm.at[cbuf.at[j]], gbuf0)
                scale(j, gbuf0)
                pltpu.sync_copy(gbuf0, acc.at[rbuf.at[j]], add=True)

        plsc.subcore_barrier()

        base = c * n_nodes
        for t in range(n_copies):
            pltpu.sync_copy(acc.at[pl.ds(s * rps + t * zr, zr)],
                            out_hbm.at[pl.ds(base + s * rps + t * zr, zr)])
        for g in range(rem_groups):
            @pl.when(s == g)
            def _():
                pltpu.sync_copy(
                    acc.at[pl.ds(NS * rps + g * 8, 8)],
                    out_hbm.at[pl.ds(base + NS * rps + g * 8, 8)])

    return sc_agg(x, rows2d, cols2d, vals2d)


def _tc_transform(acc2, W, b2, n_nodes, d):
    blk = 1000
    assert n_nodes % blk == 0

    def tc_body(acc_ref, w_ref, b_ref, o_ref):
        agg = acc_ref[0] + acc_ref[1]
        o_ref[...] = lax.dot_general(
            agg, w_ref[...], (((1,), (1,)), ((), ())),
            preferred_element_type=jnp.float32) + b_ref[...]

    return pl.pallas_call(
        tc_body,
        grid=(n_nodes // blk,),
        in_specs=[
            pl.BlockSpec((2, blk, d), lambda i: (0, i, 0)),
            pl.BlockSpec((d, d), lambda i: (0, 0)),
            pl.BlockSpec((1, d), lambda i: (0, 0)),
        ],
        out_specs=pl.BlockSpec((blk, d), lambda i: (i, 0)),
        out_shape=jax.ShapeDtypeStruct((n_nodes, d), jnp.float32),
    )(acc2, W, b2)


def kernel(x, adj_indices, adj_values, W, b):
    n_nodes, d = x.shape
    n_edges = adj_values.shape[0]

    c_edges = 80
    assert n_edges % (NW * c_edges) == 0
    n_chunks = n_edges // (NW * c_edges)
    n_chunks_pad = -(-n_chunks // 8) * 8

    def slab(a):
        a = a.reshape(NW, n_chunks * c_edges)
        pad = n_chunks_pad * c_edges - n_chunks * c_edges
        a = jnp.pad(a, ((0, 0), (0, pad)))
        return a.reshape(NW * n_chunks_pad, c_edges)

    rows2d = slab(adj_indices[0])
    cols2d = slab(adj_indices[1])
    vals2d = slab(adj_values)

    partials = _sc_aggregate(x, rows2d, cols2d, vals2d,
                             n_nodes, d, n_chunks, n_chunks_pad, c_edges)
    acc2 = partials.reshape(2, n_nodes, d)
    return _tc_transform(acc2, W, b.reshape(1, d), n_nodes, d)
